# baseline (speedup 1.0000x reference)
import functools

import jax
import jax.numpy as jnp
from jax import lax
from jax.experimental import pallas as pl
from jax.experimental.pallas import tpu as pltpu
from jax.experimental.pallas import tpu_sc as plsc

N = 10000
NP = 10112
E = 160000
D = 128
DE = 16
PAD_V = -999.0

NC = 2
NS = 16
NW = NC * NS
WN = 64
CH = 2
EH = E // CH
EPH = 81920
WPW = EPH // (NW * WN)
RPT = NP // NS

BE = 8000


def _a_body(e_ref, w_ref, b_ref, o_ref):
    e = e_ref[...]
    a = jnp.dot(e, w_ref[...], preferred_element_type=jnp.float32)
    a = a + b_ref[...]
    o_ref[...] = jnp.where(e[:, 0:1] == PAD_V, 0.0, a)


def _edge_net(e2, W, b):
    return pl.pallas_call(
        _a_body,
        grid=(EH // BE,),
        in_specs=[
            pl.BlockSpec((BE, DE), lambda i: (i, 0)),
            pl.BlockSpec((DE, D), lambda i: (0, 0)),
            pl.BlockSpec((1, D), lambda i: (0, 0)),
        ],
        out_specs=pl.BlockSpec((BE, D), lambda i: (i, 0)),
        out_shape=jax.ShapeDtypeStruct((EH, D), jnp.float32),
    )(e2, W, b.reshape(1, D))


def _sc_body(a_hbm, x_hbm, dst_hbm, src_hbm, out_hbm,
             dst_v, src_v, a_v, xs_v, xd_v, m_sh, semi, sema, semg, semsc):
    cid = lax.axis_index("c")
    sid = lax.axis_index("s")
    wid = sid * NC + cid

    def zrow(i, c):
        for j in range(D // 16):
            xs_v.at[0][i, pl.ds(j * 16, 16)] = jnp.zeros((16,), jnp.float32)
        return c

    lax.fori_loop(0, WN, zrow, 0)
    for r in range(RPT // WN):
        pltpu.sync_copy(xs_v.at[0], m_sh.at[pl.ds(sid * RPT + r * WN, WN)])
    REM = RPT % WN
    pltpu.sync_copy(xs_v.at[0].at[pl.ds(0, REM)],
                    m_sh.at[pl.ds(sid * RPT + (RPT // WN) * WN, REM)])
    plsc.subcore_barrier()

    def issue(t, p):
        base = (wid * WPW + t) * WN
        abase = jnp.minimum(base, EH - WN)
        pltpu.async_copy(dst_hbm.at[pl.ds(base, WN)], dst_v.at[p], semi[p])
        pltpu.async_copy(src_hbm.at[pl.ds(base, WN)], src_v.at[p], semi[p])
        pltpu.async_copy(a_hbm.at[pl.ds(abase, WN)], a_v.at[p], sema[p])

    def gather_issue(p):
        pltpu.make_async_copy(dst_hbm.at[pl.ds(0, WN)], dst_v.at[p], semi[p]).wait()
        pltpu.make_async_copy(src_hbm.at[pl.ds(0, WN)], src_v.at[p], semi[p]).wait()
        pltpu.async_copy(x_hbm.at[src_v.at[p]], xs_v.at[p], semg[p])
        pltpu.async_copy(x_hbm.at[dst_v.at[p]], xd_v.at[p], semg[p])

    def process(t, p):
        pltpu.make_async_copy(a_hbm.at[pl.ds(0, WN)], a_v.at[p], sema[p]).wait()
        pltpu.make_async_copy(x_hbm.at[src_v.at[p]], xs_v.at[p], semg[p]).wait()
        pltpu.make_async_copy(x_hbm.at[dst_v.at[p]], xd_v.at[p], semg[p]).wait()

        def mulrow(i, c):
            for j in range(D // 16):
                sl = pl.ds(j * 16, 16)
                av = a_v.at[p][i, sl]
                xs_v.at[p][i, sl] = xs_v.at[p][i, sl] * av
                xd_v.at[p][i, sl] = xd_v.at[p][i, sl] * av
            return c

        lax.fori_loop(0, WN, mulrow, 0)

        pltpu.sync_copy(xs_v.at[p], m_sh.at[dst_v.at[p]], add=True)
        pltpu.sync_copy(xd_v.at[p], m_sh.at[src_v.at[p]], add=True)

    issue(0, 0)
    issue(1, 1)
    gather_issue(0)

    def pair(k, carry):
        t = k * 2
        gather_issue(1)
        process(t, 0)

        @pl.when(k < WPW // 2 - 1)
        def _nxt():
            issue(t + 2, 0)
            gather_issue(0)
        process(t + 1, 1)

        @pl.when(k < WPW // 2 - 1)
        def _nxt2():
            issue(t + 3, 1)
        return carry

    lax.fori_loop(0, WPW // 2, pair, 0)
    plsc.subcore_barrier()

    pltpu.sync_copy(m_sh.at[pl.ds(sid * RPT, RPT)],
                    out_hbm.at[cid, pl.ds(sid * RPT, RPT)])


@functools.cache
def _sc_scatter_kernel():
  return functools.partial(
    pl.kernel,
    out_type=jax.ShapeDtypeStruct((NC, NP, D), jnp.float32),
    mesh=plsc.VectorSubcoreMesh(core_axis_name="c", subcore_axis_name="s",
                                num_cores=NC, num_subcores=NS),
    scratch_types=[
        pltpu.VMEM((2, WN), jnp.int32),
        pltpu.VMEM((2, WN), jnp.int32),
        pltpu.VMEM((2, WN, D), jnp.float32),
        pltpu.VMEM((2, WN, D), jnp.float32),
        pltpu.VMEM((2, WN, D), jnp.float32),
        pltpu.VMEM_SHARED((NP, D), jnp.float32),
        [pltpu.SemaphoreType.DMA, pltpu.SemaphoreType.DMA],
        [pltpu.SemaphoreType.DMA, pltpu.SemaphoreType.DMA],
        [pltpu.SemaphoreType.DMA, pltpu.SemaphoreType.DMA],
        pltpu.SemaphoreType.DMA,
    ],
  )(_sc_body)


def _sum_body(p0_ref, p1_ref, o_ref):
    o_ref[...] = (p0_ref[0] + p0_ref[1]) + (p1_ref[0] + p1_ref[1])


def _sum_partials(partial0, partial1):
    BN = 1000
    return pl.pallas_call(
        _sum_body,
        grid=(N // BN,),
        in_specs=[pl.BlockSpec((NC, BN, D), lambda i: (0, i, 0)),
                  pl.BlockSpec((NC, BN, D), lambda i: (0, i, 0))],
        out_specs=pl.BlockSpec((BN, D), lambda i: (i, 0)),
        out_shape=jax.ShapeDtypeStruct((N, D), jnp.float32),
    )(partial0, partial1)


def kernel(x, edges, pairs_idx, W, b):
    x2 = x[0]
    p2 = pairs_idx[0].astype(jnp.int32)

    xp = jnp.concatenate([x2, jnp.zeros((NP - N, D), jnp.float32)], axis=0)
    pad_idx = N + (jnp.arange(EPH - EH, dtype=jnp.int32) % (NP - N))

    parts = []
    for c in range(CH):
        pc = p2[c * EH:(c + 1) * EH]
        dst = jnp.concatenate([pc[:, 0], pad_idx])
        src = jnp.concatenate([pc[:, 1], pad_idx])
        a = _edge_net(edges[0, c * EH:(c + 1) * EH], W, b)
        parts.append(_sc_scatter_kernel()(a, xp, dst, src))
    m = _sum_partials(*parts)
    return m[None]


# speedup vs baseline: 1.1516x; 1.1516x over previous
import functools

import jax
import jax.numpy as jnp
from jax import lax
from jax.experimental import pallas as pl
from jax.experimental.pallas import tpu as pltpu
from jax.experimental.pallas import tpu_sc as plsc

N = 10000
NP = 10112
E = 160000
D = 128
DE = 16
PAD_V = -999.0

NC = 2
NS = 16
NW = NC * NS
WN = 32
EP = 163840
WPW = EP // (NW * WN)
RPT = NP // NS

BE = 8000


def _a_body(e_ref, w_ref, b_ref, o_ref):
    e = e_ref[...]
    a = jnp.dot(e, w_ref[...], preferred_element_type=jnp.float32)
    a = a + b_ref[...]
    o_ref[...] = jnp.where(e[:, 0:1] == PAD_V, 0.0, a)


def _edge_net(e2, W, b):
    return pl.pallas_call(
        _a_body,
        grid=(E // BE,),
        in_specs=[
            pl.BlockSpec((BE, DE), lambda i: (i, 0)),
            pl.BlockSpec((DE, D), lambda i: (0, 0)),
            pl.BlockSpec((1, D), lambda i: (0, 0)),
        ],
        out_specs=pl.BlockSpec((BE, D), lambda i: (i, 0)),
        out_shape=jax.ShapeDtypeStruct((E, D), jnp.float32),
    )(e2, W, b.reshape(1, D))


def _sc_body(a_hbm, x_hbm, dst_hbm, src_hbm, out_hbm,
             dst_v, src_v, sdst_v, ssrc_v, a_v, xs_v, xd_v, mos_v, mod_v,
             m_sh, semi, sema, semg, semsc):
    cid = lax.axis_index("c")
    sid = lax.axis_index("s")
    wid = sid * NC + cid

    def zrow(i, c):
        for j in range(D // 16):
            xs_v.at[0][i, pl.ds(j * 16, 16)] = jnp.zeros((16,), jnp.float32)
        return c

    lax.fori_loop(0, WN, zrow, 0)
    for r in range(RPT // WN):
        pltpu.sync_copy(xs_v.at[0], m_sh.at[pl.ds(sid * RPT + r * WN, WN)])
    REM = RPT % WN
    pltpu.sync_copy(xs_v.at[0].at[pl.ds(0, REM)],
                    m_sh.at[pl.ds(sid * RPT + (RPT // WN) * WN, REM)])
    plsc.subcore_barrier()

    def issue(t, p):
        base = (wid * WPW + t) * WN
        abase = jnp.minimum(base, E - WN)
        pltpu.async_copy(dst_hbm.at[pl.ds(base, WN)], dst_v.at[p], semi[p])
        pltpu.async_copy(src_hbm.at[pl.ds(base, WN)], src_v.at[p], semi[p])
        pltpu.async_copy(a_hbm.at[pl.ds(abase, WN)], a_v.at[p], sema[p])

    def gather_issue(p):
        pltpu.make_async_copy(dst_hbm.at[pl.ds(0, WN)], dst_v.at[p], semi[p]).wait()
        pltpu.make_async_copy(src_hbm.at[pl.ds(0, WN)], src_v.at[p], semi[p]).wait()
        pltpu.async_copy(x_hbm.at[src_v.at[p]], xs_v.at[p], semg[p])
        pltpu.async_copy(x_hbm.at[dst_v.at[p]], xd_v.at[p], semg[p])

    def process(t, p, sc_pending=True):
        if sc_pending:
            drain_scatter(p)
        else:
            pl.when(first_guard[0])(lambda: drain_scatter(p))
        pltpu.make_async_copy(a_hbm.at[pl.ds(0, WN)], a_v.at[p], sema[p]).wait()
        pltpu.make_async_copy(x_hbm.at[src_v.at[p]], xs_v.at[p], semg[p]).wait()
        pltpu.make_async_copy(x_hbm.at[dst_v.at[p]], xd_v.at[p], semg[p]).wait()

        def mulrow(i, c):
            for j in range(D // 16):
                sl = pl.ds(j * 16, 16)
                av = a_v.at[p][i, sl]
                mos_v.at[p][i, sl] = xs_v.at[p][i, sl] * av
                mod_v.at[p][i, sl] = xd_v.at[p][i, sl] * av
            return c

        lax.fori_loop(0, WN, mulrow, 0)

        for j in range(WN // 16):
            sl = pl.ds(j * 16, 16)
            sdst_v.at[p][sl] = dst_v.at[p][sl]
            ssrc_v.at[p][sl] = src_v.at[p][sl]

        pltpu.async_copy(mos_v.at[p], m_sh.at[sdst_v.at[p]], semsc[p], add=True)
        pltpu.async_copy(mod_v.at[p], m_sh.at[ssrc_v.at[p]], semsc[p], add=True)

    def drain_scatter(p):
        pltpu.make_async_copy(mos_v.at[p], m_sh.at[sdst_v.at[p]], semsc[p]).wait()
        pltpu.make_async_copy(mod_v.at[p], m_sh.at[ssrc_v.at[p]], semsc[p]).wait()

    issue(0, 0)
    issue(1, 1)
    gather_issue(0)

    first_guard = [None]

    def pair(k, carry):
        t = k * 2
        first_guard[0] = k > 0
        gather_issue(1)
        process(t, 0, sc_pending=False)

        @pl.when(k < WPW // 2 - 1)
        def _nxt():
            issue(t + 2, 0)
            gather_issue(0)
        process(t + 1, 1, sc_pending=False)

        @pl.when(k < WPW // 2 - 1)
        def _nxt2():
            issue(t + 3, 1)
        return carry

    lax.fori_loop(0, WPW // 2, pair, 0)
    drain_scatter(0)
    drain_scatter(1)
    plsc.subcore_barrier()

    pltpu.sync_copy(m_sh.at[pl.ds(sid * RPT, RPT)],
                    out_hbm.at[cid, pl.ds(sid * RPT, RPT)])


@functools.cache
def _sc_scatter_kernel():
  return functools.partial(
    pl.kernel,
    out_type=jax.ShapeDtypeStruct((NC, NP, D), jnp.float32),
    mesh=plsc.VectorSubcoreMesh(core_axis_name="c", subcore_axis_name="s",
                                num_cores=NC, num_subcores=NS),
    scratch_types=[
        pltpu.VMEM((2, WN), jnp.int32),
        pltpu.VMEM((2, WN), jnp.int32),
        pltpu.VMEM((2, WN), jnp.int32),
        pltpu.VMEM((2, WN), jnp.int32),
        pltpu.VMEM((2, WN, D), jnp.float32),
        pltpu.VMEM((2, WN, D), jnp.float32),
        pltpu.VMEM((2, WN, D), jnp.float32),
        pltpu.VMEM((2, WN, D), jnp.float32),
        pltpu.VMEM((2, WN, D), jnp.float32),
        pltpu.VMEM_SHARED((NP, D), jnp.float32),
        [pltpu.SemaphoreType.DMA, pltpu.SemaphoreType.DMA],
        [pltpu.SemaphoreType.DMA, pltpu.SemaphoreType.DMA],
        [pltpu.SemaphoreType.DMA, pltpu.SemaphoreType.DMA],
        [pltpu.SemaphoreType.DMA, pltpu.SemaphoreType.DMA],
    ],
  )(_sc_body)


def _sum_body(p_ref, o_ref):
    o_ref[...] = p_ref[0] + p_ref[1]


def _sum_partials(partial):
    BN = 1000
    return pl.pallas_call(
        _sum_body,
        grid=(N // BN,),
        in_specs=[pl.BlockSpec((NC, BN, D), lambda i: (0, i, 0))],
        out_specs=pl.BlockSpec((BN, D), lambda i: (i, 0)),
        out_shape=jax.ShapeDtypeStruct((N, D), jnp.float32),
    )(partial)


def kernel(x, edges, pairs_idx, W, b):
    x2 = x[0]
    p2 = pairs_idx[0].astype(jnp.int32)

    xp = jnp.concatenate([x2, jnp.zeros((NP - N, D), jnp.float32)], axis=0)
    pad_idx = N + (jnp.arange(EP - E, dtype=jnp.int32) % (NP - N))
    dst = jnp.concatenate([p2[:, 0], pad_idx])
    src = jnp.concatenate([p2[:, 1], pad_idx])

    a = _edge_net(edges[0], W, b)
    partial = _sc_scatter_kernel()(a, xp, dst, src)
    m = _sum_partials(partial)
    return m[None]
